# baseline (speedup 1.0000x reference)
import functools

import jax
import jax.numpy as jnp
from jax import lax
from jax.experimental import pallas as pl
from jax.experimental.pallas import tpu as pltpu
from jax.experimental.pallas import tpu_sc as plsc

_B, _S, _D = 4, 4096, 1024
_S_TC = 3584
_S_SC = _S - _S_TC
_TC_BLK = 512

_NW = 32
_S_PER_W = _S_SC // _NW
_T = 16
_CHUNK = _T * _D
_NOPS = _B
_NBUF = 4


def _tc_add(x_ref, pe_ref, o_ref):
    o_ref[0] = x_ref[0] + pe_ref[...]


def _sc_add(x_hbm, pe_hbm, out_hbm,
            pe0, o0, o1, o2, o3,
            spe0, sg0, sg1, sg2, sg3, ss0, ss1, ss2, ss3):
    wid = lax.axis_index("s") * 2 + lax.axis_index("c")
    row0 = _S_TC + wid * _S_PER_W

    o_bufs = [o0, o1, o2, o3]
    g_sems = [sg0, sg1, sg2, sg3]
    s_sems = [ss0, ss1, ss2, ss3]

    def x_off(b):
        return b * _S * _D + row0 * _D

    def out_off(b):
        return b * _S_SC * _D + (row0 - _S_TC) * _D

    pe_cp = pltpu.async_copy(
        pe_hbm.at[pl.ds(row0 * _D, _CHUNK)], pe0, spe0)
    g_cp = [pltpu.async_copy(x_hbm.at[pl.ds(x_off(b), _CHUNK)],
                             o_bufs[b], g_sems[b])
            for b in range(_NOPS)]
    pe_cp.wait()

    s_cp = [None] * _NOPS
    for b in range(_NOPS):
        g_cp[b].wait()
        ov = o_bufs[b]

        @plsc.parallel_loop(0, _CHUNK // 16, unroll=8)
        def _(i, ov=ov):
            sl = pl.ds(i * 16, 16)
            plsc.addupdate(ov.at[sl], pe0[sl])

        s_cp[b] = pltpu.async_copy(
            ov, out_hbm.at[pl.ds(out_off(b), _CHUNK)], s_sems[b])
    for b in range(_NOPS):
        s_cp[b].wait()


_sc_kernel = functools.partial(
    pl.kernel,
    mesh=plsc.VectorSubcoreMesh(core_axis_name="c", subcore_axis_name="s"),
    out_type=jax.ShapeDtypeStruct((_B * _S_SC * _D,), jnp.float32),
    scratch_types=[
        pltpu.VMEM((_CHUNK,), jnp.float32),
        pltpu.VMEM((_CHUNK,), jnp.float32),
        pltpu.VMEM((_CHUNK,), jnp.float32),
        pltpu.VMEM((_CHUNK,), jnp.float32),
        pltpu.VMEM((_CHUNK,), jnp.float32),
        pltpu.SemaphoreType.DMA,
        pltpu.SemaphoreType.DMA,
        pltpu.SemaphoreType.DMA,
        pltpu.SemaphoreType.DMA,
        pltpu.SemaphoreType.DMA,
        pltpu.SemaphoreType.DMA,
        pltpu.SemaphoreType.DMA,
        pltpu.SemaphoreType.DMA,
        pltpu.SemaphoreType.DMA,
    ],
)(_sc_add)


def kernel(x, which_dim, pos_embedding):
    del which_dim
    B, S, D = x.shape
    sc_out = _sc_kernel(x.reshape(-1), pos_embedding.reshape(-1))
    tc_out = pl.pallas_call(
        _tc_add,
        grid=(_S_TC // _TC_BLK, B),
        in_specs=[
            pl.BlockSpec((1, _TC_BLK, D), lambda i, b: (b, i, 0)),
            pl.BlockSpec((_TC_BLK, D), lambda i, b: (i, 0)),
        ],
        out_specs=pl.BlockSpec((1, _TC_BLK, D), lambda i, b: (b, i, 0)),
        out_shape=jax.ShapeDtypeStruct((B, _S_TC, D), x.dtype),
        compiler_params=pltpu.CompilerParams(
            vmem_limit_bytes=110 * 1024 * 1024,
        ),
    )(x, pos_embedding)
    return jnp.concatenate(
        [tc_out, sc_out.reshape(B, _S_SC, D)], axis=1)
